# baseline (speedup 1.0000x reference)
import functools

import jax
import jax.numpy as jnp
from jax import lax
from jax.experimental import pallas as pl
from jax.experimental.pallas import tpu as pltpu
from jax.experimental.pallas import tpu_sc as plsc

N = 10000
E = 320000
IN_DIM = 128
HID = 128
PHI = 16
NK = 4
NCLS = 40

SC_CORES = 2
SC_SUBCORES = 16
NW = SC_CORES * SC_SUBCORES
CH = 48
ITERS = 209
E_PAD = NW * ITERS * CH
NPAD = 10112
ZROWS = 632


def _make_sc_block(D):
    W = D + 16

    mesh = plsc.VectorSubcoreMesh(
        core_axis_name="c", subcore_axis_name="s",
        num_cores=SC_CORES, num_subcores=SC_SUBCORES)

    @functools.partial(
        pl.kernel,
        out_type=[
            jax.ShapeDtypeStruct((SC_CORES, NPAD, W), jnp.float32),
            jax.ShapeDtypeStruct((E_PAD, NK), jnp.float32),
        ],
        mesh=mesh,
        compiler_params=pltpu.CompilerParams(use_tc_tiling_on_sc=False,
                                             needs_layout_passes=False),
        scratch_types=[
            pltpu.VMEM((2 * CH,), jnp.int32),
            pltpu.VMEM((CH,), jnp.int32),
            pltpu.VMEM((CH,), jnp.int32),
            pltpu.VMEM((CH,), jnp.int32),
            pltpu.VMEM((CH,), jnp.int32),
            pltpu.VMEM((CH, PHI), jnp.float32),
            pltpu.VMEM((CH, PHI), jnp.float32),
            pltpu.VMEM((CH, NK), jnp.float32),
            pltpu.VMEM((CH, D), jnp.float32),
            pltpu.VMEM((CH, D), jnp.float32),
            pltpu.VMEM((CH, D), jnp.float32),
            pltpu.VMEM((CH, D), jnp.float32),
            pltpu.VMEM((CH, W), jnp.float32),
            pltpu.VMEM((5, PHI), jnp.float32),
            pltpu.VMEM_SHARED((NPAD, W), jnp.float32),
            pltpu.SemaphoreType.DMA,
        ],
    )
    def sc_block(ec_h, phi_h, hk_h, k2c_h, z_h,
                 apart_o, att_o,
                 sd, dst_v, i1, i2, i3, phis, phid, attb,
                 r0, r1, r2, r3, msg, k2cv, acc, sem):
        cid = lax.axis_index("c")
        sid = lax.axis_index("s")
        wid = cid * SC_SUBCORES + sid
        iota = lax.iota(jnp.int32, 16)
        zv = jnp.zeros((16,), jnp.float32)

        pltpu.sync_copy(k2c_h, k2cv)
        k2rows = [k2cv[k, :] for k in range(5)]
        k2s = [[k2rows[k][p] for p in range(PHI)] for k in range(NK)]
        cs = [k2rows[4][k] for k in range(NK)]

        pltpu.sync_copy(z_h, acc.at[pl.ds(sid * ZROWS, ZROWS)])
        plsc.subcore_barrier()

        tail = jnp.where(iota == 0, 1.0, 0.0).astype(jnp.float32)
        for e in range(CH):
            plsc.store_scatter(msg, [jnp.full((16,), e, jnp.int32), D + iota], tail)

        chunk0 = wid * ITERS

        def it_body(it, carry):
            chunk = chunk0 + it
            pltpu.sync_copy(ec_h.at[chunk], sd)
            for g in range(CH // 16):
                sv = sd[pl.ds(g * 16, 16)]
                i1[pl.ds(g * 16, 16)] = sv + N
                i2[pl.ds(g * 16, 16)] = sv + 2 * N
                i3[pl.ds(g * 16, 16)] = sv + 3 * N
                dst_v[pl.ds(g * 16, 16)] = sd[pl.ds(CH + g * 16, 16)]
            src_ix = sd.at[pl.ds(0, CH)]
            cps = [pltpu.async_copy(phi_h.at[src_ix], phis, sem),
                   pltpu.async_copy(phi_h.at[dst_v], phid, sem),
                   pltpu.async_copy(hk_h.at[src_ix], r0, sem),
                   pltpu.async_copy(hk_h.at[i1], r1, sem),
                   pltpu.async_copy(hk_h.at[i2], r2, sem),
                   pltpu.async_copy(hk_h.at[i3], r3, sem)]
            for cp in cps:
                cp.wait()

            for g in range(CH // 16):
                rws = g * 16 + iota
                accs = [zv, zv, zv, zv]
                for p in range(PHI):
                    pf = jnp.full((16,), p, jnp.int32)
                    ps = plsc.load_gather(phis, [rws, pf])
                    pd = plsc.load_gather(phid, [rws, pf])
                    dfp = ps - pd
                    accs = [accs[k] + dfp * k2s[k][p] for k in range(NK)]
                logits = [accs[k] - cs[k] for k in range(NK)]
                m = jnp.maximum(jnp.maximum(logits[0], logits[1]),
                                jnp.maximum(logits[2], logits[3]))
                es = [jnp.exp(l - m) for l in logits]
                inv = 1.0 / ((es[0] + es[1]) + (es[2] + es[3]))
                for k in range(NK):
                    plsc.store_scatter(attb, [rws, jnp.full((16,), k, jnp.int32)],
                                       es[k] * inv)

            @plsc.parallel_loop(0, CH, 1, unroll=4)
            def _combine(e):
                ef = jnp.full((16,), e, jnp.int32)
                w0 = plsc.load_gather(attb, [ef, jnp.full((16,), 0, jnp.int32)])
                w1 = plsc.load_gather(attb, [ef, jnp.full((16,), 1, jnp.int32)])
                w2 = plsc.load_gather(attb, [ef, jnp.full((16,), 2, jnp.int32)])
                w3 = plsc.load_gather(attb, [ef, jnp.full((16,), 3, jnp.int32)])
                for v in range(D // 16):
                    cvec = v * 16 + iota
                    mv = (w0 * plsc.load_gather(r0, [ef, cvec])
                          + w1 * plsc.load_gather(r1, [ef, cvec])
                          + w2 * plsc.load_gather(r2, [ef, cvec])
                          + w3 * plsc.load_gather(r3, [ef, cvec]))
                    plsc.store_scatter(msg, [ef, cvec], mv)

            pltpu.sync_copy(msg, acc.at[dst_v], add=True)
            pltpu.sync_copy(attb, att_o.at[pl.ds(chunk * CH, CH)])
            return carry

        lax.fori_loop(0, ITERS, it_body, 0)
        plsc.subcore_barrier()

        @pl.when(sid == 0)
        def _():
            pltpu.sync_copy(acc, apart_o.at[cid])

    return sc_block


_sc_block_hid = _make_sc_block(HID)
_sc_block_cls = _make_sc_block(NCLS + 8)


_BN = 2000


def _stage1_body(x_ref, wf_ref, bf_ref, wphi_ref, wm_ref, phi_ref, hk_ref):
    h = jnp.maximum(
        jnp.dot(x_ref[...], wf_ref[...], preferred_element_type=jnp.float32)
        + bf_ref[...], 0.0)
    phi_ref[...] = jnp.tanh(
        jnp.dot(h, wphi_ref[...], preferred_element_type=jnp.float32))
    for k in range(NK):
        hk_ref[k] = jnp.dot(h, wm_ref[k], preferred_element_type=jnp.float32)


def _stage1(x, Wf, bf2, Wphi, Wm):
    return pl.pallas_call(
        _stage1_body,
        grid=(N // _BN,),
        in_specs=[
            pl.BlockSpec((_BN, IN_DIM), lambda i: (i, 0)),
            pl.BlockSpec((IN_DIM, HID), lambda i: (0, 0)),
            pl.BlockSpec((1, HID), lambda i: (0, 0)),
            pl.BlockSpec((HID, PHI), lambda i: (0, 0)),
            pl.BlockSpec((NK, HID, HID), lambda i: (0, 0, 0)),
        ],
        out_specs=[
            pl.BlockSpec((_BN, PHI), lambda i: (i, 0)),
            pl.BlockSpec((NK, _BN, HID), lambda i: (0, i, 0)),
        ],
        out_shape=[
            jax.ShapeDtypeStruct((N, PHI), jnp.float32),
            jax.ShapeDtypeStruct((NK, N, HID), jnp.float32),
        ],
    )(x, Wf, bf2, Wphi, Wm)


def _stage2_body(ap_ref, b0_ref, wphi_ref, wm_ref, phi_ref, hk_ref):
    a = ap_ref[0] + ap_ref[1]
    deg = jnp.clip(a[:, HID:HID + 1], 1.0, None)
    h1 = jnp.maximum(a[:, :HID] / deg + b0_ref[...], 0.0)
    phi_ref[...] = jnp.tanh(
        jnp.dot(h1, wphi_ref[...], preferred_element_type=jnp.float32))
    for k in range(NK):
        hk_ref[k] = jnp.dot(h1, wm_ref[k], preferred_element_type=jnp.float32)


def _stage2(apart, b02, Wphi, Wmp):
    WA = HID + 16
    return pl.pallas_call(
        _stage2_body,
        grid=(N // _BN,),
        in_specs=[
            pl.BlockSpec((SC_CORES, _BN, WA), lambda i: (0, i, 0)),
            pl.BlockSpec((1, HID), lambda i: (0, 0)),
            pl.BlockSpec((HID, PHI), lambda i: (0, 0)),
            pl.BlockSpec((NK, HID, NCLS + 8), lambda i: (0, 0, 0)),
        ],
        out_specs=[
            pl.BlockSpec((_BN, PHI), lambda i: (i, 0)),
            pl.BlockSpec((NK, _BN, NCLS + 8), lambda i: (0, i, 0)),
        ],
        out_shape=[
            jax.ShapeDtypeStruct((N, PHI), jnp.float32),
            jax.ShapeDtypeStruct((NK, N, NCLS + 8), jnp.float32),
        ],
    )(apart, b02, Wphi, Wmp)


def _stage3_body(ap_ref, b1_ref, out_ref):
    D = NCLS + 8
    a = ap_ref[0] + ap_ref[1]
    deg = jnp.clip(a[:, D:D + 1], 1.0, None)
    logits = a[:, :D] / deg + b1_ref[...]
    col = lax.broadcasted_iota(jnp.int32, (_BN, D), 1)
    logits = jnp.where(col < NCLS, logits, -1e30)
    m = jnp.max(logits, axis=1, keepdims=True)
    x = logits - m
    lse = jnp.log(jnp.sum(jnp.exp(x), axis=1, keepdims=True))
    out_ref[...] = x - lse


def _stage3(apart, b1p):
    D = NCLS + 8
    WA = D + 16
    return pl.pallas_call(
        _stage3_body,
        grid=(N // _BN,),
        in_specs=[
            pl.BlockSpec((SC_CORES, _BN, WA), lambda i: (0, i, 0)),
            pl.BlockSpec((1, D), lambda i: (0, 0)),
        ],
        out_specs=pl.BlockSpec((_BN, D), lambda i: (i, 0)),
        out_shape=jax.ShapeDtypeStruct((N, D), jnp.float32),
    )(apart, b1p)


_LR = E * NK // 128


def _loss_body(a0_ref, a1_ref, k0_ref, k1_ref, sep_ref, foc_ref):
    i = pl.program_id(0)

    @pl.when(i == 0)
    def _():
        s = 0.0
        for kr in (k0_ref, k1_ref):
            acc = 0.0
            for a in range(NK):
                for b in range(NK):
                    d = kr[a:a + 1, :] - kr[b:b + 1, :]
                    acc = acc + jnp.exp(-jnp.sum(d * d))
            s = s + acc / (NK * NK)
        sep_ref[...] = jnp.full((1, 1), 0.5) * s
        foc_ref[...] = jnp.zeros((1, 1), jnp.float32)

    a0 = a0_ref[...]
    a1 = a1_ref[...]
    part = -(jnp.sum(a0 * jnp.log(a0 + 1e-9)) + jnp.sum(a1 * jnp.log(a1 + 1e-9)))
    foc_ref[...] = foc_ref[...] + part

    @pl.when(i == pl.num_programs(0) - 1)
    def _():
        foc_ref[...] = foc_ref[...] * (1.0 / (2.0 * E))


def _loss(a0, a1, K0, K1):
    BR = 2000
    return pl.pallas_call(
        _loss_body,
        grid=(_LR // BR,),
        in_specs=[
            pl.BlockSpec((BR, 128), lambda i: (i, 0)),
            pl.BlockSpec((BR, 128), lambda i: (i, 0)),
            pl.BlockSpec((NK, PHI), lambda i: (0, 0)),
            pl.BlockSpec((NK, PHI), lambda i: (0, 0)),
        ],
        out_specs=[
            pl.BlockSpec((1, 1), lambda i: (0, 0)),
            pl.BlockSpec((1, 1), lambda i: (0, 0)),
        ],
        out_shape=[
            jax.ShapeDtypeStruct((1, 1), jnp.float32),
            jax.ShapeDtypeStruct((1, 1), jnp.float32),
        ],
    )(a0, a1, K0, K1)


def _k2c(K):
    return jnp.concatenate(
        [2.0 * K, jnp.pad(jnp.sum(K * K, axis=-1), (0, PHI - NK)).reshape(1, PHI)], 0)


_NCHUNKS = E_PAD // CH


def kernel(features, edge_index, Wf, bf, Wphi0, K0, Wm0, b0, Wphi1, K1, Wm1, b1):
    pad = E_PAD - E
    src_p = jnp.concatenate([edge_index[0], jnp.zeros((pad,), jnp.int32)])
    dst_p = jnp.concatenate([edge_index[1], jnp.full((pad,), N, jnp.int32)])
    ec = jnp.stack([src_p.reshape(_NCHUNKS, CH), dst_p.reshape(_NCHUNKS, CH)],
                   axis=1).reshape(_NCHUNKS, 2 * CH)
    zphi = jnp.zeros((NPAD - N, PHI), jnp.float32)
    phi0, hk0 = _stage1(features, Wf, bf.reshape(1, HID), Wphi0, Wm0)
    z0 = jnp.zeros((ZROWS, HID + 16), jnp.float32)
    apart0, att0p = _sc_block_hid(ec, jnp.concatenate([phi0, zphi]),
                                  hk0.reshape(NK * N, HID), _k2c(K0), z0)
    Wm1p = jnp.pad(Wm1, ((0, 0), (0, 0), (0, 8)))
    phi1, hk1 = _stage2(apart0, b0.reshape(1, HID), Wphi1, Wm1p)
    z1 = jnp.zeros((ZROWS, NCLS + 8 + 16), jnp.float32)
    apart1, att1p = _sc_block_cls(ec, jnp.concatenate([phi1, zphi]),
                                  hk1.reshape(NK * N, NCLS + 8), _k2c(K1), z1)
    logp48 = _stage3(apart1, jnp.pad(b1, (0, 8)).reshape(1, NCLS + 8))
    logp = logp48[:, :NCLS]
    att0 = att0p[:E]
    att1 = att1p[:E]
    l_sep, l_foc = _loss(att0.reshape(_LR, 128), att1.reshape(_LR, 128), K0, K1)
    return (logp, l_sep.reshape(()), l_foc.reshape(()))


# speedup vs baseline: 1.3949x; 1.3949x over previous
import functools

import jax
import jax.numpy as jnp
from jax import lax
from jax.experimental import pallas as pl
from jax.experimental.pallas import tpu as pltpu
from jax.experimental.pallas import tpu_sc as plsc

N = 10000
E = 320000
IN_DIM = 128
HID = 128
PHI = 16
NK = 4
NCLS = 40

SC_CORES = 2
SC_SUBCORES = 16
NW = SC_CORES * SC_SUBCORES
CH = 80
ITERS = E // (NW * CH)
NPAD = 10112
ZROWS = 632


def _make_sc_block(D, two_pass):
    W = D + 16
    NRB = 2 if two_pass else 4

    mesh = plsc.VectorSubcoreMesh(
        core_axis_name="c", subcore_axis_name="s",
        num_cores=SC_CORES, num_subcores=SC_SUBCORES)

    @functools.partial(
        pl.kernel,
        out_type=[
            jax.ShapeDtypeStruct((SC_CORES, NPAD, W), jnp.float32),
            jax.ShapeDtypeStruct((E, NK), jnp.float32),
        ],
        mesh=mesh,
        compiler_params=pltpu.CompilerParams(use_tc_tiling_on_sc=False,
                                             needs_layout_passes=False),
        scratch_types=[
            pltpu.VMEM((2 * CH,), jnp.int32),
            pltpu.VMEM((CH,), jnp.int32),
            pltpu.VMEM((CH,), jnp.int32),
            pltpu.VMEM((CH,), jnp.int32),
            pltpu.VMEM((CH,), jnp.int32),
            pltpu.VMEM((CH, PHI), jnp.float32),
            pltpu.VMEM((CH, PHI), jnp.float32),
            pltpu.VMEM((CH, NK), jnp.float32),
        ] + [pltpu.VMEM((CH, D), jnp.float32)] * NRB + [
            pltpu.VMEM((CH, W), jnp.float32),
            pltpu.VMEM((5, PHI), jnp.float32),
            pltpu.VMEM_SHARED((NPAD, W), jnp.float32),
            pltpu.SemaphoreType.DMA,
            pltpu.SemaphoreType.DMA,
            pltpu.SemaphoreType.DMA,
            pltpu.SemaphoreType.DMA,
        ],
    )
    def sc_block(ec_h, phi_h, hk_h, k2c_h, z_h,
                 apart_o, att_o,
                 sd, dst_s, ia, ib, ic, phis, phid, attb,
                 *tail_refs):
        rbufs = tail_refs[:NRB]
        msg, k2cv, acc, semp, semr, semo, sema = tail_refs[NRB:]
        cid = lax.axis_index("c")
        sid = lax.axis_index("s")
        wid = cid * SC_SUBCORES + sid
        iota = lax.iota(jnp.int32, 16)
        zv = jnp.zeros((16,), jnp.float32)

        pltpu.sync_copy(k2c_h, k2cv)
        k2rows = [k2cv[k, :] for k in range(5)]
        k2s = [[k2rows[k][p] for p in range(PHI)] for k in range(NK)]
        cs = [k2rows[4][k] for k in range(NK)]

        pltpu.sync_copy(z_h, acc.at[pl.ds(sid * ZROWS, ZROWS)])
        plsc.subcore_barrier()

        tail = jnp.where(iota == 0, 1.0, 0.0).astype(jnp.float32)
        for e in range(CH):
            plsc.store_scatter(msg, [jnp.full((16,), e, jnp.int32), D + iota], tail)

        chunk0 = wid * ITERS
        src_ix = sd.at[pl.ds(0, CH)]
        dst_ix = sd.at[pl.ds(CH, CH)]

        def combine_pass(wks, rbs, accumulate):
            @plsc.parallel_loop(0, CH, 1, unroll=4)
            def _combine(e):
                ef = jnp.full((16,), e, jnp.int32)
                ws = [plsc.load_gather(attb, [ef, jnp.full((16,), k, jnp.int32)])
                      for k in wks]
                for v in range(D // 16):
                    cvec = v * 16 + iota
                    mv = ws[0] * plsc.load_gather(rbs[0], [ef, cvec])
                    for j in range(1, len(rbs)):
                        mv = mv + ws[j] * plsc.load_gather(rbs[j], [ef, cvec])
                    if accumulate:
                        mv = mv + plsc.load_gather(msg, [ef, cvec])
                    plsc.store_scatter(msg, [ef, cvec], mv)

        def it_body(it, carry):
            chunk = chunk0 + it
            pltpu.sync_copy(ec_h.at[chunk], sd)
            for g in range(CH // 16):
                sv = sd[pl.ds(g * 16, 16)]
                ia[pl.ds(g * 16, 16)] = sv + N
                if not two_pass:
                    ib[pl.ds(g * 16, 16)] = sv + 2 * N
                    ic[pl.ds(g * 16, 16)] = sv + 3 * N
            cpp = [pltpu.async_copy(phi_h.at[src_ix], phis, semp),
                   pltpu.async_copy(phi_h.at[dst_ix], phid, semp)]
            if two_pass:
                cpr = [pltpu.async_copy(hk_h.at[src_ix], rbufs[0], semr),
                       pltpu.async_copy(hk_h.at[ia], rbufs[1], semr)]
            else:
                cpr = [pltpu.async_copy(hk_h.at[src_ix], rbufs[0], semr),
                       pltpu.async_copy(hk_h.at[ia], rbufs[1], semr),
                       pltpu.async_copy(hk_h.at[ib], rbufs[2], semr),
                       pltpu.async_copy(hk_h.at[ic], rbufs[3], semr)]

            @pl.when(it > 0)
            def _():
                pltpu.make_async_copy(
                    attb, att_o.at[pl.ds((chunk - 1) * CH, CH)], sema).wait()

            for cp in cpp:
                cp.wait()

            for g in range(CH // 16):
                rws = g * 16 + iota
                accs = [zv, zv, zv, zv]
                for p in range(PHI):
                    pf = jnp.full((16,), p, jnp.int32)
                    ps = plsc.load_gather(phis, [rws, pf])
                    pd = plsc.load_gather(phid, [rws, pf])
                    dfp = ps - pd
                    accs = [accs[k] + dfp * k2s[k][p] for k in range(NK)]
                logits = [accs[k] - cs[k] for k in range(NK)]
                m = jnp.maximum(jnp.maximum(logits[0], logits[1]),
                                jnp.maximum(logits[2], logits[3]))
                es = [jnp.exp(l - m) for l in logits]
                inv = 1.0 / ((es[0] + es[1]) + (es[2] + es[3]))
                for k in range(NK):
                    plsc.store_scatter(attb, [rws, jnp.full((16,), k, jnp.int32)],
                                       es[k] * inv)

            @pl.when(it > 0)
            def _():
                pltpu.make_async_copy(msg, acc.at[dst_s], semo).wait()

            for cp in cpr:
                cp.wait()

            if two_pass:
                combine_pass((0, 1), rbufs, False)
                for g in range(CH // 16):
                    sv = sd[pl.ds(g * 16, 16)]
                    ia[pl.ds(g * 16, 16)] = sv + 2 * N
                    ib[pl.ds(g * 16, 16)] = sv + 3 * N
                cp_a = pltpu.async_copy(hk_h.at[ia], rbufs[0], semr)
                cp_b = pltpu.async_copy(hk_h.at[ib], rbufs[1], semr)
                cp_a.wait()
                cp_b.wait()
                combine_pass((2, 3), rbufs, True)
            else:
                combine_pass((0, 1, 2, 3), rbufs, False)

            for g in range(CH // 16):
                dst_s[pl.ds(g * 16, 16)] = sd[pl.ds(CH + g * 16, 16)]
            pltpu.async_copy(msg, acc.at[dst_s], semo, add=True)
            pltpu.async_copy(attb, att_o.at[pl.ds(chunk * CH, CH)], sema)
            return carry

        lax.fori_loop(0, ITERS, it_body, 0)
        pltpu.make_async_copy(msg, acc.at[dst_s], semo).wait()
        pltpu.make_async_copy(
            attb, att_o.at[pl.ds((chunk0 + ITERS - 1) * CH, CH)], sema).wait()
        plsc.subcore_barrier()

        @pl.when(sid == 0)
        def _():
            pltpu.sync_copy(acc, apart_o.at[cid])

    return sc_block


_sc_block_hid = _make_sc_block(HID, True)
_sc_block_cls = _make_sc_block(NCLS + 8, False)


_BN = 2000


def _stage1_body(x_ref, wf_ref, bf_ref, wphi_ref, wm_ref, phi_ref, hk_ref):
    h = jnp.maximum(
        jnp.dot(x_ref[...], wf_ref[...], preferred_element_type=jnp.float32)
        + bf_ref[...], 0.0)
    phi_ref[...] = jnp.tanh(
        jnp.dot(h, wphi_ref[...], preferred_element_type=jnp.float32))
    for k in range(NK):
        hk_ref[k] = jnp.dot(h, wm_ref[k], preferred_element_type=jnp.float32)


def _stage1(x, Wf, bf2, Wphi, Wm):
    return pl.pallas_call(
        _stage1_body,
        grid=(N // _BN,),
        in_specs=[
            pl.BlockSpec((_BN, IN_DIM), lambda i: (i, 0)),
            pl.BlockSpec((IN_DIM, HID), lambda i: (0, 0)),
            pl.BlockSpec((1, HID), lambda i: (0, 0)),
            pl.BlockSpec((HID, PHI), lambda i: (0, 0)),
            pl.BlockSpec((NK, HID, HID), lambda i: (0, 0, 0)),
        ],
        out_specs=[
            pl.BlockSpec((_BN, PHI), lambda i: (i, 0)),
            pl.BlockSpec((NK, _BN, HID), lambda i: (0, i, 0)),
        ],
        out_shape=[
            jax.ShapeDtypeStruct((N, PHI), jnp.float32),
            jax.ShapeDtypeStruct((NK, N, HID), jnp.float32),
        ],
    )(x, Wf, bf2, Wphi, Wm)


def _stage2_body(ap_ref, b0_ref, wphi_ref, wm_ref, phi_ref, hk_ref):
    a = ap_ref[0] + ap_ref[1]
    deg = jnp.clip(a[:, HID:HID + 1], 1.0, None)
    h1 = jnp.maximum(a[:, :HID] / deg + b0_ref[...], 0.0)
    phi_ref[...] = jnp.tanh(
        jnp.dot(h1, wphi_ref[...], preferred_element_type=jnp.float32))
    for k in range(NK):
        hk_ref[k] = jnp.dot(h1, wm_ref[k], preferred_element_type=jnp.float32)


def _stage2(apart, b02, Wphi, Wmp):
    WA = HID + 16
    return pl.pallas_call(
        _stage2_body,
        grid=(N // _BN,),
        in_specs=[
            pl.BlockSpec((SC_CORES, _BN, WA), lambda i: (0, i, 0)),
            pl.BlockSpec((1, HID), lambda i: (0, 0)),
            pl.BlockSpec((HID, PHI), lambda i: (0, 0)),
            pl.BlockSpec((NK, HID, NCLS + 8), lambda i: (0, 0, 0)),
        ],
        out_specs=[
            pl.BlockSpec((_BN, PHI), lambda i: (i, 0)),
            pl.BlockSpec((NK, _BN, NCLS + 8), lambda i: (0, i, 0)),
        ],
        out_shape=[
            jax.ShapeDtypeStruct((N, PHI), jnp.float32),
            jax.ShapeDtypeStruct((NK, N, NCLS + 8), jnp.float32),
        ],
    )(apart, b02, Wphi, Wmp)


def _stage3_body(ap_ref, b1_ref, out_ref):
    D = NCLS + 8
    a = ap_ref[0] + ap_ref[1]
    deg = jnp.clip(a[:, D:D + 1], 1.0, None)
    logits = a[:, :D] / deg + b1_ref[...]
    col = lax.broadcasted_iota(jnp.int32, (_BN, D), 1)
    logits = jnp.where(col < NCLS, logits, -1e30)
    m = jnp.max(logits, axis=1, keepdims=True)
    x = logits - m
    lse = jnp.log(jnp.sum(jnp.exp(x), axis=1, keepdims=True))
    out_ref[...] = x - lse


def _stage3(apart, b1p):
    D = NCLS + 8
    WA = D + 16
    return pl.pallas_call(
        _stage3_body,
        grid=(N // _BN,),
        in_specs=[
            pl.BlockSpec((SC_CORES, _BN, WA), lambda i: (0, i, 0)),
            pl.BlockSpec((1, D), lambda i: (0, 0)),
        ],
        out_specs=pl.BlockSpec((_BN, D), lambda i: (i, 0)),
        out_shape=jax.ShapeDtypeStruct((N, D), jnp.float32),
    )(apart, b1p)


_LR = E * NK // 128


def _loss_body(a0_ref, a1_ref, k0_ref, k1_ref, sep_ref, foc_ref):
    i = pl.program_id(0)

    @pl.when(i == 0)
    def _():
        s = 0.0
        for kr in (k0_ref, k1_ref):
            acc = 0.0
            for a in range(NK):
                for b in range(NK):
                    d = kr[a:a + 1, :] - kr[b:b + 1, :]
                    acc = acc + jnp.exp(-jnp.sum(d * d))
            s = s + acc / (NK * NK)
        sep_ref[...] = jnp.full((1, 1), 0.5) * s
        foc_ref[...] = jnp.zeros((1, 1), jnp.float32)

    a0 = a0_ref[...]
    a1 = a1_ref[...]
    part = -(jnp.sum(a0 * jnp.log(a0 + 1e-9)) + jnp.sum(a1 * jnp.log(a1 + 1e-9)))
    foc_ref[...] = foc_ref[...] + part

    @pl.when(i == pl.num_programs(0) - 1)
    def _():
        foc_ref[...] = foc_ref[...] * (1.0 / (2.0 * E))


def _loss(a0, a1, K0, K1):
    BR = 2000
    return pl.pallas_call(
        _loss_body,
        grid=(_LR // BR,),
        in_specs=[
            pl.BlockSpec((BR, 128), lambda i: (i, 0)),
            pl.BlockSpec((BR, 128), lambda i: (i, 0)),
            pl.BlockSpec((NK, PHI), lambda i: (0, 0)),
            pl.BlockSpec((NK, PHI), lambda i: (0, 0)),
        ],
        out_specs=[
            pl.BlockSpec((1, 1), lambda i: (0, 0)),
            pl.BlockSpec((1, 1), lambda i: (0, 0)),
        ],
        out_shape=[
            jax.ShapeDtypeStruct((1, 1), jnp.float32),
            jax.ShapeDtypeStruct((1, 1), jnp.float32),
        ],
    )(a0, a1, K0, K1)


def _k2c(K):
    return jnp.concatenate(
        [2.0 * K, jnp.pad(jnp.sum(K * K, axis=-1), (0, PHI - NK)).reshape(1, PHI)], 0)


_NCHUNKS = E // CH


def kernel(features, edge_index, Wf, bf, Wphi0, K0, Wm0, b0, Wphi1, K1, Wm1, b1):
    ec = jnp.stack([edge_index[0].reshape(_NCHUNKS, CH),
                    edge_index[1].reshape(_NCHUNKS, CH)],
                   axis=1).reshape(_NCHUNKS, 2 * CH)
    phi0, hk0 = _stage1(features, Wf, bf.reshape(1, HID), Wphi0, Wm0)
    z0 = jnp.zeros((ZROWS, HID + 16), jnp.float32)
    apart0, att0 = _sc_block_hid(ec, phi0, hk0.reshape(NK * N, HID),
                                 _k2c(K0), z0)
    Wm1p = jnp.pad(Wm1, ((0, 0), (0, 0), (0, 8)))
    phi1, hk1 = _stage2(apart0, b0.reshape(1, HID), Wphi1, Wm1p)
    z1 = jnp.zeros((ZROWS, NCLS + 8 + 16), jnp.float32)
    apart1, att1 = _sc_block_cls(ec, phi1, hk1.reshape(NK * N, NCLS + 8),
                                 _k2c(K1), z1)
    logp48 = _stage3(apart1, jnp.pad(b1, (0, 8)).reshape(1, NCLS + 8))
    logp = logp48[:, :NCLS]
    l_sep, l_foc = _loss(att0.reshape(_LR, 128), att1.reshape(_LR, 128), K0, K1)
    return (logp, l_sep.reshape(()), l_foc.reshape(()))


# speedup vs baseline: 1.5657x; 1.1224x over previous
import functools

import jax
import jax.numpy as jnp
from jax import lax
from jax.experimental import pallas as pl
from jax.experimental.pallas import tpu as pltpu
from jax.experimental.pallas import tpu_sc as plsc

N = 10000
E = 320000
IN_DIM = 128
HID = 128
PHI = 16
NK = 4
NCLS = 40

SC_CORES = 2
SC_SUBCORES = 16
NW = SC_CORES * SC_SUBCORES
CH = 80
ITERS = E // (NW * CH)
NPAD = 10112
ZROWS = 632


def _make_sc_block(D, two_pass):
    W = D + 16
    NRB = 2 if two_pass else 4

    mesh = plsc.VectorSubcoreMesh(
        core_axis_name="c", subcore_axis_name="s",
        num_cores=SC_CORES, num_subcores=SC_SUBCORES)

    @functools.partial(
        pl.kernel,
        out_type=[
            jax.ShapeDtypeStruct((SC_CORES, NPAD, W), jnp.float32),
            jax.ShapeDtypeStruct((E, NK), jnp.float32),
        ],
        mesh=mesh,
        compiler_params=pltpu.CompilerParams(use_tc_tiling_on_sc=False,
                                             needs_layout_passes=False),
        scratch_types=[
            pltpu.VMEM((2 * CH,), jnp.int32),
            pltpu.VMEM((2 * CH,), jnp.int32),
            pltpu.VMEM((CH,), jnp.int32),
            pltpu.VMEM((CH,), jnp.int32),
            pltpu.VMEM((CH,), jnp.int32),
            pltpu.VMEM((CH,), jnp.int32),
            pltpu.VMEM((CH, PHI), jnp.float32),
            pltpu.VMEM((CH, PHI), jnp.float32),
            pltpu.VMEM((CH, NK), jnp.float32),
        ] + [pltpu.VMEM((CH, D), jnp.float32)] * NRB + [
            pltpu.VMEM((CH, W), jnp.float32),
            pltpu.VMEM((5, PHI), jnp.float32),
            pltpu.VMEM_SHARED((NPAD, W), jnp.float32),
            pltpu.SemaphoreType.DMA,
            pltpu.SemaphoreType.DMA,
            pltpu.SemaphoreType.DMA,
            pltpu.SemaphoreType.DMA,
            pltpu.SemaphoreType.DMA,
        ],
    )
    def sc_block(ec_h, phi_h, hk_h, k2c_h, z_h,
                 apart_o, att_o,
                 sd, sd_nx, dst_s, ia, ib, ic, phis, phid, attb,
                 *tail_refs):
        rbufs = tail_refs[:NRB]
        msg, k2cv, acc, semp, semr, semo, sema, sems = tail_refs[NRB:]
        cid = lax.axis_index("c")
        sid = lax.axis_index("s")
        wid = cid * SC_SUBCORES + sid
        iota = lax.iota(jnp.int32, 16)
        zv = jnp.zeros((16,), jnp.float32)

        pltpu.sync_copy(k2c_h, k2cv)
        k2rows = [k2cv[k, :] for k in range(5)]
        k2s = [[k2rows[k][p] for p in range(PHI)] for k in range(NK)]
        cs = [k2rows[4][k] for k in range(NK)]

        pltpu.sync_copy(z_h, acc.at[pl.ds(sid * ZROWS, ZROWS)])
        plsc.subcore_barrier()

        tail = jnp.where(iota == 0, 1.0, 0.0).astype(jnp.float32)
        for e in range(CH):
            plsc.store_scatter(msg, [jnp.full((16,), e, jnp.int32), D + iota], tail)

        chunk0 = wid * ITERS
        src_ix = sd.at[pl.ds(0, CH)]
        dst_ix = sd.at[pl.ds(CH, CH)]

        def build_k_idx(ks, irefs_):
            for g in range(CH // 16):
                sv = sd[pl.ds(g * 16, 16)]
                for k, ir in zip(ks, irefs_):
                    ir[pl.ds(g * 16, 16)] = sv + k * N

        def fire_first_rows():
            if two_pass:
                build_k_idx((1,), (ia,))
                pltpu.async_copy(hk_h.at[src_ix], rbufs[0], semr)
                pltpu.async_copy(hk_h.at[ia], rbufs[1], semr)
            else:
                build_k_idx((1, 2, 3), (ia, ib, ic))
                pltpu.async_copy(hk_h.at[src_ix], rbufs[0], semr)
                pltpu.async_copy(hk_h.at[ia], rbufs[1], semr)
                pltpu.async_copy(hk_h.at[ib], rbufs[2], semr)
                pltpu.async_copy(hk_h.at[ic], rbufs[3], semr)

        def wait_phi():
            pltpu.make_async_copy(phi_h.at[src_ix], phis, semp).wait()
            pltpu.make_async_copy(phi_h.at[dst_ix], phid, semp).wait()

        def wait_first_rows():
            for j in range(NRB):
                pltpu.make_async_copy(hk_h.at[src_ix], rbufs[j], semr).wait()

        def combine_pass(wks, rbs, accumulate):
            @plsc.parallel_loop(0, CH, 1, unroll=4)
            def _combine(e):
                ef = jnp.full((16,), e, jnp.int32)
                ws = [plsc.load_gather(attb, [ef, jnp.full((16,), k, jnp.int32)])
                      for k in wks]
                for v in range(D // 16):
                    cvec = v * 16 + iota
                    mv = ws[0] * plsc.load_gather(rbs[0], [ef, cvec])
                    for j in range(1, len(rbs)):
                        mv = mv + ws[j] * plsc.load_gather(rbs[j], [ef, cvec])
                    if accumulate:
                        mv = mv + plsc.load_gather(msg, [ef, cvec])
                    plsc.store_scatter(msg, [ef, cvec], mv)

        def it_body(it, carry):
            chunk = chunk0 + it

            @pl.when(it > 0)
            def _():
                pltpu.make_async_copy(
                    attb, att_o.at[pl.ds((chunk - 1) * CH, CH)], sema).wait()

            wait_phi()

            nxt = jnp.minimum(chunk + 1, (E // CH) - 1)
            pltpu.async_copy(ec_h.at[nxt], sd_nx, sems)

            for g in range(CH // 16):
                rws = g * 16 + iota
                accs = [zv, zv, zv, zv]
                for p in range(PHI):
                    pf = jnp.full((16,), p, jnp.int32)
                    ps = plsc.load_gather(phis, [rws, pf])
                    pd = plsc.load_gather(phid, [rws, pf])
                    dfp = ps - pd
                    accs = [accs[k] + dfp * k2s[k][p] for k in range(NK)]
                logits = [accs[k] - cs[k] for k in range(NK)]
                m = jnp.maximum(jnp.maximum(logits[0], logits[1]),
                                jnp.maximum(logits[2], logits[3]))
                es = [jnp.exp(l - m) for l in logits]
                inv = 1.0 / ((es[0] + es[1]) + (es[2] + es[3]))
                for k in range(NK):
                    plsc.store_scatter(attb, [rws, jnp.full((16,), k, jnp.int32)],
                                       es[k] * inv)

            pltpu.async_copy(attb, att_o.at[pl.ds(chunk * CH, CH)], sema)

            pltpu.make_async_copy(ec_h.at[nxt], sd_nx, sems).wait()
            pltpu.async_copy(phi_h.at[sd_nx.at[pl.ds(0, CH)]], phis, semp)
            pltpu.async_copy(phi_h.at[sd_nx.at[pl.ds(CH, CH)]], phid, semp)

            @pl.when(it > 0)
            def _():
                pltpu.make_async_copy(msg, acc.at[dst_s], semo).wait()

            wait_first_rows()

            if two_pass:
                combine_pass((0, 1), rbufs, False)
                for g in range(CH // 16):
                    sv = sd[pl.ds(g * 16, 16)]
                    ia[pl.ds(g * 16, 16)] = sv + 2 * N
                    ib[pl.ds(g * 16, 16)] = sv + 3 * N
                cp_a = pltpu.async_copy(hk_h.at[ia], rbufs[0], semr)
                cp_b = pltpu.async_copy(hk_h.at[ib], rbufs[1], semr)
                cp_a.wait()
                cp_b.wait()
                combine_pass((2, 3), rbufs, True)
            else:
                combine_pass((0, 1, 2, 3), rbufs, False)

            for g in range(CH // 16):
                dst_s[pl.ds(g * 16, 16)] = sd[pl.ds(CH + g * 16, 16)]
            pltpu.async_copy(msg, acc.at[dst_s], semo, add=True)

            for g in range(2 * CH // 16):
                sd[pl.ds(g * 16, 16)] = sd_nx[pl.ds(g * 16, 16)]
            fire_first_rows()
            return carry

        pltpu.sync_copy(ec_h.at[chunk0], sd)
        pltpu.async_copy(phi_h.at[src_ix], phis, semp)
        pltpu.async_copy(phi_h.at[dst_ix], phid, semp)
        fire_first_rows()

        lax.fori_loop(0, ITERS, it_body, 0)
        pltpu.make_async_copy(msg, acc.at[dst_s], semo).wait()
        pltpu.make_async_copy(
            attb, att_o.at[pl.ds((chunk0 + ITERS - 1) * CH, CH)], sema).wait()
        wait_phi()
        wait_first_rows()
        plsc.subcore_barrier()

        @pl.when(sid == 0)
        def _():
            pltpu.sync_copy(acc, apart_o.at[cid])

    return sc_block


_sc_block_hid = _make_sc_block(HID, True)
_sc_block_cls = _make_sc_block(NCLS + 8, False)


_BN = 2000


def _stage1_body(x_ref, wf_ref, bf_ref, wphi_ref, wm_ref, phi_ref, hk_ref):
    h = jnp.maximum(
        jnp.dot(x_ref[...], wf_ref[...], preferred_element_type=jnp.float32)
        + bf_ref[...], 0.0)
    phi_ref[...] = jnp.tanh(
        jnp.dot(h, wphi_ref[...], preferred_element_type=jnp.float32))
    for k in range(NK):
        hk_ref[k] = jnp.dot(h, wm_ref[k], preferred_element_type=jnp.float32)


def _stage1(x, Wf, bf2, Wphi, Wm):
    return pl.pallas_call(
        _stage1_body,
        grid=(N // _BN,),
        in_specs=[
            pl.BlockSpec((_BN, IN_DIM), lambda i: (i, 0)),
            pl.BlockSpec((IN_DIM, HID), lambda i: (0, 0)),
            pl.BlockSpec((1, HID), lambda i: (0, 0)),
            pl.BlockSpec((HID, PHI), lambda i: (0, 0)),
            pl.BlockSpec((NK, HID, HID), lambda i: (0, 0, 0)),
        ],
        out_specs=[
            pl.BlockSpec((_BN, PHI), lambda i: (i, 0)),
            pl.BlockSpec((NK, _BN, HID), lambda i: (0, i, 0)),
        ],
        out_shape=[
            jax.ShapeDtypeStruct((N, PHI), jnp.float32),
            jax.ShapeDtypeStruct((NK, N, HID), jnp.float32),
        ],
    )(x, Wf, bf2, Wphi, Wm)


def _stage2_body(ap_ref, b0_ref, wphi_ref, wm_ref, phi_ref, hk_ref):
    a = ap_ref[0] + ap_ref[1]
    deg = jnp.clip(a[:, HID:HID + 1], 1.0, None)
    h1 = jnp.maximum(a[:, :HID] / deg + b0_ref[...], 0.0)
    phi_ref[...] = jnp.tanh(
        jnp.dot(h1, wphi_ref[...], preferred_element_type=jnp.float32))
    for k in range(NK):
        hk_ref[k] = jnp.dot(h1, wm_ref[k], preferred_element_type=jnp.float32)


def _stage2(apart, b02, Wphi, Wmp):
    WA = HID + 16
    return pl.pallas_call(
        _stage2_body,
        grid=(N // _BN,),
        in_specs=[
            pl.BlockSpec((SC_CORES, _BN, WA), lambda i: (0, i, 0)),
            pl.BlockSpec((1, HID), lambda i: (0, 0)),
            pl.BlockSpec((HID, PHI), lambda i: (0, 0)),
            pl.BlockSpec((NK, HID, NCLS + 8), lambda i: (0, 0, 0)),
        ],
        out_specs=[
            pl.BlockSpec((_BN, PHI), lambda i: (i, 0)),
            pl.BlockSpec((NK, _BN, NCLS + 8), lambda i: (0, i, 0)),
        ],
        out_shape=[
            jax.ShapeDtypeStruct((N, PHI), jnp.float32),
            jax.ShapeDtypeStruct((NK, N, NCLS + 8), jnp.float32),
        ],
    )(apart, b02, Wphi, Wmp)


def _stage3_body(ap_ref, b1_ref, out_ref):
    D = NCLS + 8
    a = ap_ref[0] + ap_ref[1]
    deg = jnp.clip(a[:, D:D + 1], 1.0, None)
    logits = a[:, :D] / deg + b1_ref[...]
    col = lax.broadcasted_iota(jnp.int32, (_BN, D), 1)
    logits = jnp.where(col < NCLS, logits, -1e30)
    m = jnp.max(logits, axis=1, keepdims=True)
    x = logits - m
    lse = jnp.log(jnp.sum(jnp.exp(x), axis=1, keepdims=True))
    out_ref[...] = x - lse


def _stage3(apart, b1p):
    D = NCLS + 8
    WA = D + 16
    return pl.pallas_call(
        _stage3_body,
        grid=(N // _BN,),
        in_specs=[
            pl.BlockSpec((SC_CORES, _BN, WA), lambda i: (0, i, 0)),
            pl.BlockSpec((1, D), lambda i: (0, 0)),
        ],
        out_specs=pl.BlockSpec((_BN, D), lambda i: (i, 0)),
        out_shape=jax.ShapeDtypeStruct((N, D), jnp.float32),
    )(apart, b1p)


_LR = E * NK // 128


def _loss_body(a0_ref, a1_ref, k0_ref, k1_ref, sep_ref, foc_ref):
    i = pl.program_id(0)

    @pl.when(i == 0)
    def _():
        s = 0.0
        for kr in (k0_ref, k1_ref):
            acc = 0.0
            for a in range(NK):
                for b in range(NK):
                    d = kr[a:a + 1, :] - kr[b:b + 1, :]
                    acc = acc + jnp.exp(-jnp.sum(d * d))
            s = s + acc / (NK * NK)
        sep_ref[...] = jnp.full((1, 1), 0.5) * s
        foc_ref[...] = jnp.zeros((1, 1), jnp.float32)

    a0 = a0_ref[...]
    a1 = a1_ref[...]
    part = -(jnp.sum(a0 * jnp.log(a0 + 1e-9)) + jnp.sum(a1 * jnp.log(a1 + 1e-9)))
    foc_ref[...] = foc_ref[...] + part

    @pl.when(i == pl.num_programs(0) - 1)
    def _():
        foc_ref[...] = foc_ref[...] * (1.0 / (2.0 * E))


def _loss(a0, a1, K0, K1):
    BR = 2000
    return pl.pallas_call(
        _loss_body,
        grid=(_LR // BR,),
        in_specs=[
            pl.BlockSpec((BR, 128), lambda i: (i, 0)),
            pl.BlockSpec((BR, 128), lambda i: (i, 0)),
            pl.BlockSpec((NK, PHI), lambda i: (0, 0)),
            pl.BlockSpec((NK, PHI), lambda i: (0, 0)),
        ],
        out_specs=[
            pl.BlockSpec((1, 1), lambda i: (0, 0)),
            pl.BlockSpec((1, 1), lambda i: (0, 0)),
        ],
        out_shape=[
            jax.ShapeDtypeStruct((1, 1), jnp.float32),
            jax.ShapeDtypeStruct((1, 1), jnp.float32),
        ],
    )(a0, a1, K0, K1)


def _k2c(K):
    return jnp.concatenate(
        [2.0 * K, jnp.pad(jnp.sum(K * K, axis=-1), (0, PHI - NK)).reshape(1, PHI)], 0)


_NCHUNKS = E // CH


def kernel(features, edge_index, Wf, bf, Wphi0, K0, Wm0, b0, Wphi1, K1, Wm1, b1):
    ec = jnp.stack([edge_index[0].reshape(_NCHUNKS, CH),
                    edge_index[1].reshape(_NCHUNKS, CH)],
                   axis=1).reshape(_NCHUNKS, 2 * CH)
    phi0, hk0 = _stage1(features, Wf, bf.reshape(1, HID), Wphi0, Wm0)
    z0 = jnp.zeros((ZROWS, HID + 16), jnp.float32)
    apart0, att0 = _sc_block_hid(ec, phi0, hk0.reshape(NK * N, HID),
                                 _k2c(K0), z0)
    Wm1p = jnp.pad(Wm1, ((0, 0), (0, 0), (0, 8)))
    phi1, hk1 = _stage2(apart0, b0.reshape(1, HID), Wphi1, Wm1p)
    z1 = jnp.zeros((ZROWS, NCLS + 8 + 16), jnp.float32)
    apart1, att1 = _sc_block_cls(ec, phi1, hk1.reshape(NK * N, NCLS + 8),
                                 _k2c(K1), z1)
    logp48 = _stage3(apart1, jnp.pad(b1, (0, 8)).reshape(1, NCLS + 8))
    logp = logp48[:, :NCLS]
    l_sep, l_foc = _loss(att0.reshape(_LR, 128), att1.reshape(_LR, 128), K0, K1)
    return (logp, l_sep.reshape(()), l_foc.reshape(()))


# speedup vs baseline: 1.5898x; 1.0154x over previous
import functools

import jax
import jax.numpy as jnp
from jax import lax
from jax.experimental import pallas as pl
from jax.experimental.pallas import tpu as pltpu
from jax.experimental.pallas import tpu_sc as plsc

N = 10000
E = 320000
IN_DIM = 128
HID = 128
PHI = 16
NK = 4
NCLS = 40

SC_CORES = 2
SC_SUBCORES = 16
NW = SC_CORES * SC_SUBCORES
CH = 80
ITERS = E // (NW * CH)
NPAD = 10112
ZROWS = 632


def _make_sc_block(D, packed):
    W = D + 16
    NRB = 4
    rb_type = (pltpu.VMEM((CH, D // 2), jnp.int32) if packed
               else pltpu.VMEM((CH, D), jnp.float32))

    mesh = plsc.VectorSubcoreMesh(
        core_axis_name="c", subcore_axis_name="s",
        num_cores=SC_CORES, num_subcores=SC_SUBCORES)

    @functools.partial(
        pl.kernel,
        out_type=[
            jax.ShapeDtypeStruct((SC_CORES, NPAD, W), jnp.float32),
            jax.ShapeDtypeStruct((E, NK), jnp.float32),
        ],
        mesh=mesh,
        compiler_params=pltpu.CompilerParams(use_tc_tiling_on_sc=False,
                                             needs_layout_passes=False),
        scratch_types=[
            pltpu.VMEM((2 * CH,), jnp.int32),
            pltpu.VMEM((2 * CH,), jnp.int32),
            pltpu.VMEM((CH,), jnp.int32),
            pltpu.VMEM((CH,), jnp.int32),
            pltpu.VMEM((CH,), jnp.int32),
            pltpu.VMEM((CH,), jnp.int32),
            pltpu.VMEM((CH, PHI), jnp.float32),
            pltpu.VMEM((CH, PHI), jnp.float32),
            pltpu.VMEM((CH, NK), jnp.float32),
        ] + [rb_type] * NRB + [
            pltpu.VMEM((CH, W), jnp.float32),
            pltpu.VMEM((5, PHI), jnp.float32),
            pltpu.VMEM_SHARED((NPAD, W), jnp.float32),
            pltpu.SemaphoreType.DMA,
            pltpu.SemaphoreType.DMA,
            pltpu.SemaphoreType.DMA,
            pltpu.SemaphoreType.DMA,
            pltpu.SemaphoreType.DMA,
        ],
    )
    def sc_block(ec_h, phi_h, hk_h, k2c_h, z_h,
                 apart_o, att_o,
                 sd, sd_nx, dst_s, ia, ib, ic, phis, phid, attb,
                 *tail_refs):
        rbufs = tail_refs[:NRB]
        msg, k2cv, acc, semp, semr, semo, sema, sems = tail_refs[NRB:]
        cid = lax.axis_index("c")
        sid = lax.axis_index("s")
        wid = cid * SC_SUBCORES + sid
        iota = lax.iota(jnp.int32, 16)
        zv = jnp.zeros((16,), jnp.float32)

        pltpu.sync_copy(k2c_h, k2cv)
        k2rows = [k2cv[k, :] for k in range(5)]
        k2s = [[k2rows[k][p] for p in range(PHI)] for k in range(NK)]
        cs = [k2rows[4][k] for k in range(NK)]

        pltpu.sync_copy(z_h, acc.at[pl.ds(sid * ZROWS, ZROWS)])
        plsc.subcore_barrier()

        tail = jnp.where(iota == 0, 1.0, 0.0).astype(jnp.float32)
        for e in range(CH):
            plsc.store_scatter(msg, [jnp.full((16,), e, jnp.int32), D + iota], tail)

        chunk0 = wid * ITERS
        src_ix = sd.at[pl.ds(0, CH)]
        dst_ix = sd.at[pl.ds(CH, CH)]

        def build_k_idx(ks, irefs_):
            for g in range(CH // 16):
                sv = sd[pl.ds(g * 16, 16)]
                for k, ir in zip(ks, irefs_):
                    ir[pl.ds(g * 16, 16)] = sv + k * N

        def fire_first_rows():
            build_k_idx((1, 2, 3), (ia, ib, ic))
            pltpu.async_copy(hk_h.at[src_ix], rbufs[0], semr)
            pltpu.async_copy(hk_h.at[ia], rbufs[1], semr)
            pltpu.async_copy(hk_h.at[ib], rbufs[2], semr)
            pltpu.async_copy(hk_h.at[ic], rbufs[3], semr)

        def wait_phi():
            pltpu.make_async_copy(phi_h.at[src_ix], phis, semp).wait()
            pltpu.make_async_copy(phi_h.at[dst_ix], phid, semp).wait()

        def wait_first_rows():
            for j in range(NRB):
                pltpu.make_async_copy(hk_h.at[src_ix], rbufs[j], semr).wait()

        def combine_all():
            @plsc.parallel_loop(0, CH, 1, unroll=4)
            def _combine(e):
                ef = jnp.full((16,), e, jnp.int32)
                ws = [plsc.load_gather(attb, [ef, jnp.full((16,), k, jnp.int32)])
                      for k in range(NK)]
                if packed:
                    for v in range(D // 32):
                        cvec = v * 16 + iota
                        ma = None
                        mb = None
                        for k in range(NK):
                            pk = plsc.load_gather(rbufs[k], [ef, cvec])
                            ak, bk = plsc.unpack(
                                plsc.bitcast(pk, jnp.bfloat16),
                                format=plsc.PackFormat.INTERLEAVED,
                                preferred_element_type=jnp.float32)
                            ma = ws[k] * ak if ma is None else ma + ws[k] * ak
                            mb = ws[k] * bk if mb is None else mb + ws[k] * bk
                        ca = 32 * v + 2 * iota
                        plsc.store_scatter(msg, [ef, ca], ma)
                        plsc.store_scatter(msg, [ef, ca + 1], mb)
                else:
                    for v in range(D // 16):
                        cvec = v * 16 + iota
                        mv = ws[0] * plsc.load_gather(rbufs[0], [ef, cvec])
                        for j in range(1, NK):
                            mv = mv + ws[j] * plsc.load_gather(rbufs[j], [ef, cvec])
                        plsc.store_scatter(msg, [ef, cvec], mv)

        def it_body(it, carry):
            chunk = chunk0 + it

            @pl.when(it > 0)
            def _():
                pltpu.make_async_copy(
                    attb, att_o.at[pl.ds((chunk - 1) * CH, CH)], sema).wait()

            wait_phi()

            nxt = jnp.minimum(chunk + 1, (E // CH) - 1)
            pltpu.async_copy(ec_h.at[nxt], sd_nx, sems)

            for g in range(CH // 16):
                rws = g * 16 + iota
                accs = [zv, zv, zv, zv]
                for p in range(PHI):
                    pf = jnp.full((16,), p, jnp.int32)
                    ps = plsc.load_gather(phis, [rws, pf])
                    pd = plsc.load_gather(phid, [rws, pf])
                    dfp = ps - pd
                    accs = [accs[k] + dfp * k2s[k][p] for k in range(NK)]
                logits = [accs[k] - cs[k] for k in range(NK)]
                m = jnp.maximum(jnp.maximum(logits[0], logits[1]),
                                jnp.maximum(logits[2], logits[3]))
                es = [jnp.exp(l - m) for l in logits]
                inv = 1.0 / ((es[0] + es[1]) + (es[2] + es[3]))
                for k in range(NK):
                    plsc.store_scatter(attb, [rws, jnp.full((16,), k, jnp.int32)],
                                       es[k] * inv)

            pltpu.async_copy(attb, att_o.at[pl.ds(chunk * CH, CH)], sema)

            pltpu.make_async_copy(ec_h.at[nxt], sd_nx, sems).wait()
            pltpu.async_copy(phi_h.at[sd_nx.at[pl.ds(0, CH)]], phis, semp)
            pltpu.async_copy(phi_h.at[sd_nx.at[pl.ds(CH, CH)]], phid, semp)

            @pl.when(it > 0)
            def _():
                pltpu.make_async_copy(msg, acc.at[dst_s], semo).wait()

            wait_first_rows()
            combine_all()

            for g in range(CH // 16):
                dst_s[pl.ds(g * 16, 16)] = sd[pl.ds(CH + g * 16, 16)]
            pltpu.async_copy(msg, acc.at[dst_s], semo, add=True)

            for g in range(2 * CH // 16):
                sd[pl.ds(g * 16, 16)] = sd_nx[pl.ds(g * 16, 16)]
            fire_first_rows()
            return carry

        pltpu.sync_copy(ec_h.at[chunk0], sd)
        pltpu.async_copy(phi_h.at[src_ix], phis, semp)
        pltpu.async_copy(phi_h.at[dst_ix], phid, semp)
        fire_first_rows()

        lax.fori_loop(0, ITERS, it_body, 0)
        pltpu.make_async_copy(msg, acc.at[dst_s], semo).wait()
        pltpu.make_async_copy(
            attb, att_o.at[pl.ds((chunk0 + ITERS - 1) * CH, CH)], sema).wait()
        wait_phi()
        wait_first_rows()
        plsc.subcore_barrier()

        @pl.when(sid == 0)
        def _():
            pltpu.sync_copy(acc, apart_o.at[cid])

    return sc_block


_sc_block_hid = _make_sc_block(HID, True)
_sc_block_cls = _make_sc_block(NCLS + 8, False)


def _pack_bf16(x):
    return jax.lax.bitcast_convert_type(
        x.reshape(x.shape[0], x.shape[1] // 2, 2), jnp.int32)


_BN = 2000


def _stage1_body(x_ref, wf_ref, bf_ref, wphi_ref, wm_ref, phi_ref, hk_ref):
    h = jnp.maximum(
        jnp.dot(x_ref[...], wf_ref[...], preferred_element_type=jnp.float32)
        + bf_ref[...], 0.0)
    phi_ref[...] = jnp.tanh(
        jnp.dot(h, wphi_ref[...], preferred_element_type=jnp.float32))
    for k in range(NK):
        hk_ref[k] = jnp.dot(
            h, wm_ref[k], preferred_element_type=jnp.float32
        ).astype(jnp.bfloat16)


def _stage1(x, Wf, bf2, Wphi, Wm):
    return pl.pallas_call(
        _stage1_body,
        grid=(N // _BN,),
        in_specs=[
            pl.BlockSpec((_BN, IN_DIM), lambda i: (i, 0)),
            pl.BlockSpec((IN_DIM, HID), lambda i: (0, 0)),
            pl.BlockSpec((1, HID), lambda i: (0, 0)),
            pl.BlockSpec((HID, PHI), lambda i: (0, 0)),
            pl.BlockSpec((NK, HID, HID), lambda i: (0, 0, 0)),
        ],
        out_specs=[
            pl.BlockSpec((_BN, PHI), lambda i: (i, 0)),
            pl.BlockSpec((NK, _BN, HID), lambda i: (0, i, 0)),
        ],
        out_shape=[
            jax.ShapeDtypeStruct((N, PHI), jnp.float32),
            jax.ShapeDtypeStruct((NK, N, HID), jnp.bfloat16),
        ],
    )(x, Wf, bf2, Wphi, Wm)


def _stage2_body(ap_ref, b0_ref, wphi_ref, wm_ref, phi_ref, hk_ref):
    a = ap_ref[0] + ap_ref[1]
    deg = jnp.clip(a[:, HID:HID + 1], 1.0, None)
    h1 = jnp.maximum(a[:, :HID] / deg + b0_ref[...], 0.0)
    phi_ref[...] = jnp.tanh(
        jnp.dot(h1, wphi_ref[...], preferred_element_type=jnp.float32))
    for k in range(NK):
        hk_ref[k] = jnp.dot(h1, wm_ref[k], preferred_element_type=jnp.float32)


def _stage2(apart, b02, Wphi, Wmp):
    WA = HID + 16
    return pl.pallas_call(
        _stage2_body,
        grid=(N // _BN,),
        in_specs=[
            pl.BlockSpec((SC_CORES, _BN, WA), lambda i: (0, i, 0)),
            pl.BlockSpec((1, HID), lambda i: (0, 0)),
            pl.BlockSpec((HID, PHI), lambda i: (0, 0)),
            pl.BlockSpec((NK, HID, NCLS + 8), lambda i: (0, 0, 0)),
        ],
        out_specs=[
            pl.BlockSpec((_BN, PHI), lambda i: (i, 0)),
            pl.BlockSpec((NK, _BN, NCLS + 8), lambda i: (0, i, 0)),
        ],
        out_shape=[
            jax.ShapeDtypeStruct((N, PHI), jnp.float32),
            jax.ShapeDtypeStruct((NK, N, NCLS + 8), jnp.float32),
        ],
    )(apart, b02, Wphi, Wmp)


def _stage3_body(ap_ref, b1_ref, out_ref):
    D = NCLS + 8
    a = ap_ref[0] + ap_ref[1]
    deg = jnp.clip(a[:, D:D + 1], 1.0, None)
    logits = a[:, :D] / deg + b1_ref[...]
    col = lax.broadcasted_iota(jnp.int32, (_BN, D), 1)
    logits = jnp.where(col < NCLS, logits, -1e30)
    m = jnp.max(logits, axis=1, keepdims=True)
    x = logits - m
    lse = jnp.log(jnp.sum(jnp.exp(x), axis=1, keepdims=True))
    out_ref[...] = x - lse


def _stage3(apart, b1p):
    D = NCLS + 8
    WA = D + 16
    return pl.pallas_call(
        _stage3_body,
        grid=(N // _BN,),
        in_specs=[
            pl.BlockSpec((SC_CORES, _BN, WA), lambda i: (0, i, 0)),
            pl.BlockSpec((1, D), lambda i: (0, 0)),
        ],
        out_specs=pl.BlockSpec((_BN, D), lambda i: (i, 0)),
        out_shape=jax.ShapeDtypeStruct((N, D), jnp.float32),
    )(apart, b1p)


_LR = E * NK // 128


def _loss_body(a0_ref, a1_ref, k0_ref, k1_ref, sep_ref, foc_ref):
    i = pl.program_id(0)

    @pl.when(i == 0)
    def _():
        s = 0.0
        for kr in (k0_ref, k1_ref):
            acc = 0.0
            for a in range(NK):
                for b in range(NK):
                    d = kr[a:a + 1, :] - kr[b:b + 1, :]
                    acc = acc + jnp.exp(-jnp.sum(d * d))
            s = s + acc / (NK * NK)
        sep_ref[...] = jnp.full((1, 1), 0.5) * s
        foc_ref[...] = jnp.zeros((1, 1), jnp.float32)

    a0 = a0_ref[...]
    a1 = a1_ref[...]
    part = -(jnp.sum(a0 * jnp.log(a0 + 1e-9)) + jnp.sum(a1 * jnp.log(a1 + 1e-9)))
    foc_ref[...] = foc_ref[...] + part

    @pl.when(i == pl.num_programs(0) - 1)
    def _():
        foc_ref[...] = foc_ref[...] * (1.0 / (2.0 * E))


def _loss(a0, a1, K0, K1):
    BR = 2000
    return pl.pallas_call(
        _loss_body,
        grid=(_LR // BR,),
        in_specs=[
            pl.BlockSpec((BR, 128), lambda i: (i, 0)),
            pl.BlockSpec((BR, 128), lambda i: (i, 0)),
            pl.BlockSpec((NK, PHI), lambda i: (0, 0)),
            pl.BlockSpec((NK, PHI), lambda i: (0, 0)),
        ],
        out_specs=[
            pl.BlockSpec((1, 1), lambda i: (0, 0)),
            pl.BlockSpec((1, 1), lambda i: (0, 0)),
        ],
        out_shape=[
            jax.ShapeDtypeStruct((1, 1), jnp.float32),
            jax.ShapeDtypeStruct((1, 1), jnp.float32),
        ],
    )(a0, a1, K0, K1)


def _k2c(K):
    return jnp.concatenate(
        [2.0 * K, jnp.pad(jnp.sum(K * K, axis=-1), (0, PHI - NK)).reshape(1, PHI)], 0)


_NCHUNKS = E // CH


def kernel(features, edge_index, Wf, bf, Wphi0, K0, Wm0, b0, Wphi1, K1, Wm1, b1):
    ec = jnp.stack([edge_index[0].reshape(_NCHUNKS, CH),
                    edge_index[1].reshape(_NCHUNKS, CH)],
                   axis=1).reshape(_NCHUNKS, 2 * CH)
    phi0, hk0 = _stage1(features, Wf, bf.reshape(1, HID), Wphi0, Wm0)
    z0 = jnp.zeros((ZROWS, HID + 16), jnp.float32)
    apart0, att0 = _sc_block_hid(ec, phi0,
                                 _pack_bf16(hk0.reshape(NK * N, HID)),
                                 _k2c(K0), z0)
    Wm1p = jnp.pad(Wm1, ((0, 0), (0, 0), (0, 8)))
    phi1, hk1 = _stage2(apart0, b0.reshape(1, HID), Wphi1, Wm1p)
    z1 = jnp.zeros((ZROWS, NCLS + 8 + 16), jnp.float32)
    apart1, att1 = _sc_block_cls(ec, phi1, hk1.reshape(NK * N, NCLS + 8),
                                 _k2c(K1), z1)
    logp48 = _stage3(apart1, jnp.pad(b1, (0, 8)).reshape(1, NCLS + 8))
    logp = logp48[:, :NCLS]
    l_sep, l_foc = _loss(att0.reshape(_LR, 128), att1.reshape(_LR, 128), K0, K1)
    return (logp, l_sep.reshape(()), l_foc.reshape(()))
